# Initial kernel scaffold; baseline (speedup 1.0000x reference)
import jax
import jax.numpy as jnp
from jax.experimental import pallas as pl


def kernel(p0, p1, p2, targets):
    raise NotImplementedError("write your pallas kernel here")


# speedup vs baseline: 1.2444x; 1.2444x over previous
import functools

import jax
import jax.numpy as jnp
import numpy as np
from jax import lax
from jax.experimental import pallas as pl
from jax.experimental.pallas import tpu as pltpu
from jax.experimental.pallas import tpu_sc as plsc

_NC = 80
_NA = 3
_B = 16
_NT = 320
_NCAND = 5120
_CPAD = 88
_HWS = ((80, 80), (40, 40), (20, 20))
_STRIDES = np.array([8.0, 16.0, 32.0], dtype=np.float32)
_ANCH = (np.array(
    [[10, 13, 16, 30, 33, 23],
     [30, 61, 62, 45, 59, 119],
     [116, 90, 156, 198, 373, 326]], dtype=np.float32)
    .reshape(3, 3, 2) / _STRIDES.reshape(3, 1, 1))
_BAL = (4.0, 1.0, 0.4)
_KEYBASE = (0, 0, 81920)
_DUMPBASE = (307200, 76800, 101120)
_SBUF = 312320
_EPL = _CPAD * _NCAND
_EPT = _EPL // 32
_ROWS = _EPT // 128

_sc_info = None


def _prep_kernel(tT_ref, meta_ref, idx_ref, keys_ref):
    r = lax.broadcasted_iota(jnp.int32, (16, _NT), 0)
    o = r // 3
    a = r % 3
    valid = jnp.where(r < 15, 1.0, 0.0).astype(jnp.float32)
    bf = tT_ref[0:1, :]
    clsf = tT_ref[1:2, :]
    for l in range(3):
        H, W = _HWS[l]
        hw = H * W
        xx = tT_ref[2:3, :] * float(W) + jnp.zeros((16, _NT), jnp.float32)
        yy = tT_ref[3:4, :] * float(H) + jnp.zeros((16, _NT), jnp.float32)
        ww = tT_ref[4:5, :] * float(W) + jnp.zeros((16, _NT), jnp.float32)
        hh = tT_ref[5:6, :] * float(H) + jnp.zeros((16, _NT), jnp.float32)
        aw = jnp.where(a == 0, float(_ANCH[l, 0, 0]),
             jnp.where(a == 1, float(_ANCH[l, 1, 0]), float(_ANCH[l, 2, 0])))
        ah = jnp.where(a == 0, float(_ANCH[l, 0, 1]),
             jnp.where(a == 1, float(_ANCH[l, 1, 1]), float(_ANCH[l, 2, 1])))
        rw = ww / aw
        rh = hh / ah
        mr = jnp.maximum(jnp.maximum(rw, 1.0 / rw), jnp.maximum(rh, 1.0 / rh))
        mself = jnp.where(mr < 4.0, 1.0, 0.0)
        remx = xx - jnp.floor(xx)
        remy = yy - jnp.floor(yy)
        one = jnp.ones((16, _NT), jnp.float32)
        zero = jnp.zeros((16, _NT), jnp.float32)
        lxc = jnp.where(remx < 0.5, one, zero) * jnp.where(xx > 1.0, one, zero)
        lyc = jnp.where(remy < 0.5, one, zero) * jnp.where(yy > 1.0, one, zero)
        gxc = (jnp.where(remx > 0.5, one, zero)
               * jnp.where(xx < float(W) - 1.0, one, zero))
        gyc = (jnp.where(remy > 0.5, one, zero)
               * jnp.where(yy < float(H) - 1.0, one, zero))
        cond = jnp.where(o == 0, one,
               jnp.where(o == 1, lxc,
               jnp.where(o == 2, lyc,
               jnp.where(o == 3, gxc, gyc))))
        maskf = mself * cond * valid
        offx = jnp.where(o == 1, 0.5, jnp.where(o == 3, -0.5, 0.0))
        offy = jnp.where(o == 2, 0.5, jnp.where(o == 4, -0.5, 0.0))
        gridx = jnp.clip((xx - offx).astype(jnp.int32), 0, W - 1)
        gridy = jnp.clip((yy - offy).astype(jnp.int32), 0, H - 1)
        bi = bf.astype(jnp.int32) + jnp.zeros((16, _NT), jnp.int32)
        meta_ref[l, 0] = maskf
        meta_ref[l, 1] = xx - gridx.astype(jnp.float32)
        meta_ref[l, 2] = yy - gridy.astype(jnp.float32)
        meta_ref[l, 3] = ww
        meta_ref[l, 4] = hh
        meta_ref[l, 5] = aw + jnp.zeros((16, _NT), jnp.float32)
        meta_ref[l, 6] = ah + jnp.zeros((16, _NT), jnp.float32)
        meta_ref[l, 7] = clsf + jnp.zeros((16, _NT), jnp.float32)
        base = ((bi * 255 + a * 85) * H + gridy) * W + gridx
        cix = lax.broadcasted_iota(jnp.int32, (_CPAD, 16, _NT), 0)
        idx_ref[l] = jnp.where(cix < 85, base[None, :, :] + cix * hw, 0)
        flat = r * _NT + lax.broadcasted_iota(jnp.int32, (16, _NT), 1)
        pos = ((bi * 3 + a) * H + gridy) * W + gridx + _KEYBASE[l]
        keys_ref[l] = jnp.where(maskf > 0.0, pos, _DUMPBASE[l] + flat)


def _dense_kernel(p0_ref, p1_ref, p2_ref, out_ref):
    b = pl.program_id(0)
    a = pl.program_id(1)

    @pl.when((b == 0) & (a == 0))
    def _():
        for l in range(3):
            out_ref[l] = 0.0

    for l, ref in enumerate((p0_ref, p1_ref, p2_ref)):
        x = ref[0, 0]
        s = jnp.sum(jnp.maximum(x, 0.0) + jnp.log1p(jnp.exp(-jnp.abs(x))))
        out_ref[l] += s


def _sc_body(pf0, pf1, pf2, idx_hbm, keys_hbm, gath_hbm, win_hbm,
             idxv, gbuf, kv, idv, wv, sbuf, sem):
    c = lax.axis_index("c")
    s = lax.axis_index("s")
    wid = s * 2 + c
    for l, pf in enumerate((pf0, pf1, pf2)):
        pltpu.sync_copy(
            idx_hbm.at[pl.ds(l * _EPL + wid * _EPT, _EPT)], idxv)

        def _chunk(j, carry, pf=pf):
            descs = [
                pltpu.async_copy(
                    pf.at[idxv.at[pl.ds((j * 10 + i) * 128, 128)]],
                    gbuf.at[pl.ds((j * 10 + i) * 128, 128)], sem)
                for i in range(10)]
            for d in descs:
                d.wait()
            return carry

        lax.fori_loop(0, _ROWS // 10, _chunk, 0)
        pltpu.sync_copy(
            gbuf, gath_hbm.at[pl.ds(l * _EPL + wid * _EPT, _EPT)])
    for q in range(4):
        for k in range(5):
            idv[q, pl.ds(k * 16, 16)] = (
                s * 320 + q * 80 + k * 16 + lax.iota(jnp.int32, 16))
    my_layers = ((0,), (1, 2))
    for ci in range(2):
        @pl.when(c == ci)
        def _():
            for l in my_layers[ci]:
                for j in range(4):
                    pltpu.sync_copy(
                        keys_hbm.at[pl.ds(l * _NCAND + s * 320 + j * 80, 80)],
                        kv.at[j])
                for j in range(4):
                    pltpu.sync_copy(idv.at[j], sbuf.at[kv.at[j]])
    plsc.subcore_barrier()
    for ci in range(2):
        @pl.when(c == ci)
        def _():
            for l in my_layers[ci]:
                for j in range(4):
                    pltpu.sync_copy(
                        keys_hbm.at[pl.ds(l * _NCAND + s * 320 + j * 80, 80)],
                        kv.at[j])
                for j in range(4):
                    pltpu.sync_copy(sbuf.at[kv.at[j]], wv.at[j])
                for j in range(4):
                    pltpu.sync_copy(
                        wv.at[j],
                        win_hbm.at[pl.ds(l * _NCAND + s * 320 + j * 80, 80)])


_sc_gather = functools.partial(
    pl.kernel,
    out_type=(jax.ShapeDtypeStruct((3 * _EPL,), jnp.float32),
              jax.ShapeDtypeStruct((3 * _NCAND,), jnp.int32)),
    mesh=plsc.VectorSubcoreMesh(core_axis_name="c", subcore_axis_name="s"),
    scratch_types=(
        pltpu.VMEM((_EPT,), jnp.int32),
        pltpu.VMEM((_EPT,), jnp.float32),
        pltpu.VMEM((4, 80), jnp.int32),
        pltpu.VMEM((4, 80), jnp.int32),
        pltpu.VMEM((4, 80), jnp.int32),
        pltpu.VMEM_SHARED((_SBUF,), jnp.int32),
        pltpu.SemaphoreType.DMA,
    ),
)(_sc_body)


def _final_kernel(gath_ref, meta_ref, win_ref, dense_ref, out_ref):
    lbox = 0.0
    lobj = 0.0
    lcls = 0.0
    cid = lax.broadcasted_iota(jnp.int32, (1, _NCAND), 1)
    crow = lax.broadcasted_iota(jnp.int32, (_NC, _NCAND), 0)
    for l in range(3):
        H, W = _HWS[l]
        mask = meta_ref[l, 0:1, :]
        gxr = meta_ref[l, 1:2, :]
        gyr = meta_ref[l, 2:3, :]
        gw = meta_ref[l, 3:4, :]
        gh = meta_ref[l, 4:5, :]
        aw = meta_ref[l, 5:6, :]
        ah = meta_ref[l, 6:7, :]
        clsf = meta_ref[l, 7:8, :]
        px = jax.nn.sigmoid(gath_ref[l, 0:1, :]) * 2.0 - 0.5
        py = jax.nn.sigmoid(gath_ref[l, 1:2, :]) * 2.0 - 0.5
        pw = jnp.square(jax.nn.sigmoid(gath_ref[l, 2:3, :]) * 2.0) * aw
        ph = jnp.square(jax.nn.sigmoid(gath_ref[l, 3:4, :]) * 2.0) * ah
        a_xmin = px - (pw - 1.0) / 2.0
        a_xmax = px + (pw - 1.0) / 2.0
        a_ymin = py - (ph - 1.0) / 2.0
        a_ymax = py + (ph - 1.0) / 2.0
        b_xmin = gxr - (gw - 1.0) / 2.0
        b_xmax = gxr + (gw - 1.0) / 2.0
        b_ymin = gyr - (gh - 1.0) / 2.0
        b_ymax = gyr + (gh - 1.0) / 2.0
        iw = jnp.clip(jnp.minimum(a_xmax, b_xmax)
                      - jnp.maximum(a_xmin, b_xmin) + 1.0, 0.0)
        ih = jnp.clip(jnp.minimum(a_ymax, b_ymax)
                      - jnp.maximum(a_ymin, b_ymin) + 1.0, 0.0)
        inter = iw * ih
        union = pw * ph + gw * gh - inter
        iou = inter / union
        cw = jnp.maximum(a_xmax, b_xmax) - jnp.minimum(a_xmin, b_xmin) + 1.0
        ch = jnp.maximum(a_ymax, b_ymax) - jnp.minimum(a_ymin, b_ymin) + 1.0
        ca = cw * ch
        giou = iou - (ca - union) / ca
        cnt = jnp.sum(mask)
        box_sum = jnp.sum(mask * (1.0 - giou))
        lbox += jnp.where(cnt > 0.0, box_sum / jnp.maximum(cnt, 1.0), 0.0)
        lobj_logit = gath_ref[l, 4:5, :]
        winm = ((win_ref[l:l + 1, :] == cid) & (mask > 0.0)).astype(jnp.float32)
        corr = jnp.sum(winm * lobj_logit * jnp.clip(giou, 0.0))
        lobj += (dense_ref[l] - corr) / float(_B * _NA * H * W) * _BAL[l]
        ocls = gath_ref[l, 5:85, :]
        ct = (crow == clsf.astype(jnp.int32)).astype(jnp.float32)
        bce = (jnp.maximum(ocls, 0.0) - ocls * ct
               + jnp.log1p(jnp.exp(-jnp.abs(ocls))))
        cls_sum = jnp.sum(bce * mask)
        lcls += jnp.where(cnt > 0.0,
                          cls_sum / (jnp.maximum(cnt, 1.0) * float(_NC)), 0.0)
    out_ref[0] = (lbox * 0.05 + lobj * 1.0 + lcls * 0.5) * float(_B)


def kernel(p0, p1, p2, targets):
    tT = targets.T
    meta, idx, keys = pl.pallas_call(
        _prep_kernel,
        out_shape=(
            jax.ShapeDtypeStruct((3, 8, 16, _NT), jnp.float32),
            jax.ShapeDtypeStruct((3, _CPAD, 16, _NT), jnp.int32),
            jax.ShapeDtypeStruct((3, 16, _NT), jnp.int32),
        ),
    )(tT)
    dense = pl.pallas_call(
        _dense_kernel,
        grid=(16, 3),
        in_specs=[
            pl.BlockSpec((1, 1, 80, 80), lambda b, a: (b, a * 85 + 4, 0, 0)),
            pl.BlockSpec((1, 1, 40, 40), lambda b, a: (b, a * 85 + 4, 0, 0)),
            pl.BlockSpec((1, 1, 20, 20), lambda b, a: (b, a * 85 + 4, 0, 0)),
        ],
        out_specs=pl.BlockSpec(memory_space=pltpu.SMEM),
        out_shape=jax.ShapeDtypeStruct((3,), jnp.float32),
    )(p0, p1, p2)
    gath, win = _sc_gather(
        p0.reshape(-1), p1.reshape(-1), p2.reshape(-1),
        idx.reshape(-1),
        keys.reshape(-1),
    )
    loss = pl.pallas_call(
        _final_kernel,
        in_specs=[
            pl.BlockSpec((3, _CPAD, _NCAND), lambda: (0, 0, 0)),
            pl.BlockSpec((3, 8, _NCAND), lambda: (0, 0, 0)),
            pl.BlockSpec((3, _NCAND), lambda: (0, 0)),
            pl.BlockSpec(memory_space=pltpu.SMEM),
        ],
        out_specs=pl.BlockSpec(memory_space=pltpu.SMEM),
        out_shape=jax.ShapeDtypeStruct((1,), jnp.float32),
    )(gath.reshape(3, _CPAD, _NCAND), meta.reshape(3, 8, _NCAND),
      win.reshape(3, _NCAND), dense)
    return loss[0]


# speedup vs baseline: 1.6239x; 1.3050x over previous
import functools

import jax
import jax.numpy as jnp
import numpy as np
from jax import lax
from jax.experimental import pallas as pl
from jax.experimental.pallas import tpu as pltpu
from jax.experimental.pallas import tpu_sc as plsc

_NC = 80
_NA = 3
_B = 16
_NT = 320
_NCAND = 5120
_CPAD = 88
_HWS = ((80, 80), (40, 40), (20, 20))
_STRIDES = np.array([8.0, 16.0, 32.0], dtype=np.float32)
_ANCH = (np.array(
    [[10, 13, 16, 30, 33, 23],
     [30, 61, 62, 45, 59, 119],
     [116, 90, 156, 198, 373, 326]], dtype=np.float32)
    .reshape(3, 3, 2) / _STRIDES.reshape(3, 1, 1))
_BAL = (4.0, 1.0, 0.4)
_KEYBASE = (0, 0, 81920)
_DUMPBASE = (307200, 76800, 101120)
_SBUF = 312320
_EPL = _CPAD * _NCAND
_EPT = _EPL // 32
_ROWS = _EPT // 128

_sc_info = None


def _prep_kernel(tT_ref, meta_ref, idx_ref, keys_ref):
    r = lax.broadcasted_iota(jnp.int32, (16, _NT), 0)
    o = r // 3
    a = r % 3
    valid = jnp.where(r < 15, 1.0, 0.0).astype(jnp.float32)
    bf = tT_ref[0:1, :]
    clsf = tT_ref[1:2, :]
    for l in range(3):
        H, W = _HWS[l]
        hw = H * W
        xx = tT_ref[2:3, :] * float(W) + jnp.zeros((16, _NT), jnp.float32)
        yy = tT_ref[3:4, :] * float(H) + jnp.zeros((16, _NT), jnp.float32)
        ww = tT_ref[4:5, :] * float(W) + jnp.zeros((16, _NT), jnp.float32)
        hh = tT_ref[5:6, :] * float(H) + jnp.zeros((16, _NT), jnp.float32)
        aw = jnp.where(a == 0, float(_ANCH[l, 0, 0]),
             jnp.where(a == 1, float(_ANCH[l, 1, 0]), float(_ANCH[l, 2, 0])))
        ah = jnp.where(a == 0, float(_ANCH[l, 0, 1]),
             jnp.where(a == 1, float(_ANCH[l, 1, 1]), float(_ANCH[l, 2, 1])))
        rw = ww / aw
        rh = hh / ah
        mr = jnp.maximum(jnp.maximum(rw, 1.0 / rw), jnp.maximum(rh, 1.0 / rh))
        mself = jnp.where(mr < 4.0, 1.0, 0.0)
        remx = xx - jnp.floor(xx)
        remy = yy - jnp.floor(yy)
        one = jnp.ones((16, _NT), jnp.float32)
        zero = jnp.zeros((16, _NT), jnp.float32)
        lxc = jnp.where(remx < 0.5, one, zero) * jnp.where(xx > 1.0, one, zero)
        lyc = jnp.where(remy < 0.5, one, zero) * jnp.where(yy > 1.0, one, zero)
        gxc = (jnp.where(remx > 0.5, one, zero)
               * jnp.where(xx < float(W) - 1.0, one, zero))
        gyc = (jnp.where(remy > 0.5, one, zero)
               * jnp.where(yy < float(H) - 1.0, one, zero))
        cond = jnp.where(o == 0, one,
               jnp.where(o == 1, lxc,
               jnp.where(o == 2, lyc,
               jnp.where(o == 3, gxc, gyc))))
        maskf = mself * cond * valid
        offx = jnp.where(o == 1, 0.5, jnp.where(o == 3, -0.5, 0.0))
        offy = jnp.where(o == 2, 0.5, jnp.where(o == 4, -0.5, 0.0))
        gridx = jnp.clip((xx - offx).astype(jnp.int32), 0, W - 1)
        gridy = jnp.clip((yy - offy).astype(jnp.int32), 0, H - 1)
        bi = bf.astype(jnp.int32) + jnp.zeros((16, _NT), jnp.int32)
        meta_ref[l, 0] = maskf
        meta_ref[l, 1] = xx - gridx.astype(jnp.float32)
        meta_ref[l, 2] = yy - gridy.astype(jnp.float32)
        meta_ref[l, 3] = ww
        meta_ref[l, 4] = hh
        meta_ref[l, 5] = aw + jnp.zeros((16, _NT), jnp.float32)
        meta_ref[l, 6] = ah + jnp.zeros((16, _NT), jnp.float32)
        meta_ref[l, 7] = clsf + jnp.zeros((16, _NT), jnp.float32)
        base = ((bi * 255 + a * 85) * H + gridy) * W + gridx
        cix = lax.broadcasted_iota(jnp.int32, (_CPAD, 16, _NT), 0)
        cm = jnp.where(cix < 85, cix, cix - 85)
        idx_ref[l] = base[None, :, :] + cm * hw
        flat = r * _NT + lax.broadcasted_iota(jnp.int32, (16, _NT), 1)
        pos = ((bi * 3 + a) * H + gridy) * W + gridx + _KEYBASE[l]
        keys_ref[l] = jnp.where(maskf > 0.0, pos, _DUMPBASE[l] + flat)


def _dense_kernel(p0_ref, p1_ref, p2_ref, out_ref):
    b = pl.program_id(0)
    a = pl.program_id(1)

    @pl.when((b == 0) & (a == 0))
    def _():
        for l in range(3):
            out_ref[l] = 0.0

    for l, ref in enumerate((p0_ref, p1_ref, p2_ref)):
        x = ref[0, 0]
        s = jnp.sum(jnp.maximum(x, 0.0) + jnp.log1p(jnp.exp(-jnp.abs(x))))
        out_ref[l] += s


def _sc_body(pf0, pf1, pf2, idx_hbm, keys_hbm, gath_hbm, win_hbm,
             idxv, gbuf, kv, idv, wv, sbuf, sem):
    c = lax.axis_index("c")
    s = lax.axis_index("s")
    wid = s * 2 + c
    for l, pf in enumerate((pf0, pf1, pf2)):
        pltpu.sync_copy(
            idx_hbm.at[pl.ds(l * _EPL + wid * _EPT, _EPT)], idxv)
        pltpu.async_copy(pf.at[idxv], gbuf, sem).wait()
        pltpu.sync_copy(
            gbuf, gath_hbm.at[pl.ds(l * _EPL + wid * _EPT, _EPT)])
    for q in range(4):
        for k in range(5):
            idv[q, pl.ds(k * 16, 16)] = (
                s * 320 + q * 80 + k * 16 + lax.iota(jnp.int32, 16))
    my_layers = ((0,), (1, 2))
    for ci in range(2):
        @pl.when(c == ci)
        def _():
            for l in my_layers[ci]:
                for j in range(4):
                    pltpu.sync_copy(
                        keys_hbm.at[pl.ds(l * _NCAND + s * 320 + j * 80, 80)],
                        kv.at[j])
                for j in range(4):
                    pltpu.sync_copy(idv.at[j], sbuf.at[kv.at[j]])
    plsc.subcore_barrier()
    for ci in range(2):
        @pl.when(c == ci)
        def _():
            for l in my_layers[ci]:
                for j in range(4):
                    pltpu.sync_copy(
                        keys_hbm.at[pl.ds(l * _NCAND + s * 320 + j * 80, 80)],
                        kv.at[j])
                for j in range(4):
                    pltpu.sync_copy(sbuf.at[kv.at[j]], wv.at[j])
                for j in range(4):
                    pltpu.sync_copy(
                        wv.at[j],
                        win_hbm.at[pl.ds(l * _NCAND + s * 320 + j * 80, 80)])


_sc_gather = functools.partial(
    pl.kernel,
    out_type=(jax.ShapeDtypeStruct((3 * _EPL,), jnp.float32),
              jax.ShapeDtypeStruct((3 * _NCAND,), jnp.int32)),
    mesh=plsc.VectorSubcoreMesh(core_axis_name="c", subcore_axis_name="s"),
    scratch_types=(
        pltpu.VMEM((_EPT,), jnp.int32),
        pltpu.VMEM((_EPT,), jnp.float32),
        pltpu.VMEM((4, 80), jnp.int32),
        pltpu.VMEM((4, 80), jnp.int32),
        pltpu.VMEM((4, 80), jnp.int32),
        pltpu.VMEM_SHARED((_SBUF,), jnp.int32),
        pltpu.SemaphoreType.DMA,
    ),
)(_sc_body)


def _final_kernel(gath_ref, meta_ref, win_ref, dense_ref, out_ref):
    lbox = 0.0
    lobj = 0.0
    lcls = 0.0
    cid = lax.broadcasted_iota(jnp.int32, (1, _NCAND), 1)
    crow = lax.broadcasted_iota(jnp.int32, (_NC, _NCAND), 0)
    for l in range(3):
        H, W = _HWS[l]
        mask = meta_ref[l, 0:1, :]
        gxr = meta_ref[l, 1:2, :]
        gyr = meta_ref[l, 2:3, :]
        gw = meta_ref[l, 3:4, :]
        gh = meta_ref[l, 4:5, :]
        aw = meta_ref[l, 5:6, :]
        ah = meta_ref[l, 6:7, :]
        clsf = meta_ref[l, 7:8, :]
        px = jax.nn.sigmoid(gath_ref[l, 0:1, :]) * 2.0 - 0.5
        py = jax.nn.sigmoid(gath_ref[l, 1:2, :]) * 2.0 - 0.5
        pw = jnp.square(jax.nn.sigmoid(gath_ref[l, 2:3, :]) * 2.0) * aw
        ph = jnp.square(jax.nn.sigmoid(gath_ref[l, 3:4, :]) * 2.0) * ah
        a_xmin = px - (pw - 1.0) / 2.0
        a_xmax = px + (pw - 1.0) / 2.0
        a_ymin = py - (ph - 1.0) / 2.0
        a_ymax = py + (ph - 1.0) / 2.0
        b_xmin = gxr - (gw - 1.0) / 2.0
        b_xmax = gxr + (gw - 1.0) / 2.0
        b_ymin = gyr - (gh - 1.0) / 2.0
        b_ymax = gyr + (gh - 1.0) / 2.0
        iw = jnp.clip(jnp.minimum(a_xmax, b_xmax)
                      - jnp.maximum(a_xmin, b_xmin) + 1.0, 0.0)
        ih = jnp.clip(jnp.minimum(a_ymax, b_ymax)
                      - jnp.maximum(a_ymin, b_ymin) + 1.0, 0.0)
        inter = iw * ih
        union = pw * ph + gw * gh - inter
        iou = inter / union
        cw = jnp.maximum(a_xmax, b_xmax) - jnp.minimum(a_xmin, b_xmin) + 1.0
        ch = jnp.maximum(a_ymax, b_ymax) - jnp.minimum(a_ymin, b_ymin) + 1.0
        ca = cw * ch
        giou = iou - (ca - union) / ca
        cnt = jnp.sum(mask)
        box_sum = jnp.sum(mask * (1.0 - giou))
        lbox += jnp.where(cnt > 0.0, box_sum / jnp.maximum(cnt, 1.0), 0.0)
        lobj_logit = gath_ref[l, 4:5, :]
        winm = ((win_ref[l:l + 1, :] == cid) & (mask > 0.0)).astype(jnp.float32)
        corr = jnp.sum(winm * lobj_logit * jnp.clip(giou, 0.0))
        lobj += (dense_ref[l] - corr) / float(_B * _NA * H * W) * _BAL[l]
        ocls = gath_ref[l, 5:85, :]
        ct = (crow == clsf.astype(jnp.int32)).astype(jnp.float32)
        bce = (jnp.maximum(ocls, 0.0) - ocls * ct
               + jnp.log1p(jnp.exp(-jnp.abs(ocls))))
        cls_sum = jnp.sum(bce * mask)
        lcls += jnp.where(cnt > 0.0,
                          cls_sum / (jnp.maximum(cnt, 1.0) * float(_NC)), 0.0)
    out_ref[0] = (lbox * 0.05 + lobj * 1.0 + lcls * 0.5) * float(_B)


def kernel(p0, p1, p2, targets):
    tT = targets.T
    meta, idx, keys = pl.pallas_call(
        _prep_kernel,
        out_shape=(
            jax.ShapeDtypeStruct((3, 8, 16, _NT), jnp.float32),
            jax.ShapeDtypeStruct((3, _CPAD, 16, _NT), jnp.int32),
            jax.ShapeDtypeStruct((3, 16, _NT), jnp.int32),
        ),
    )(tT)
    dense = pl.pallas_call(
        _dense_kernel,
        grid=(16, 3),
        in_specs=[
            pl.BlockSpec((1, 1, 80, 80), lambda b, a: (b, a * 85 + 4, 0, 0)),
            pl.BlockSpec((1, 1, 40, 40), lambda b, a: (b, a * 85 + 4, 0, 0)),
            pl.BlockSpec((1, 1, 20, 20), lambda b, a: (b, a * 85 + 4, 0, 0)),
        ],
        out_specs=pl.BlockSpec(memory_space=pltpu.SMEM),
        out_shape=jax.ShapeDtypeStruct((3,), jnp.float32),
    )(p0, p1, p2)
    gath, win = _sc_gather(
        p0.reshape(-1), p1.reshape(-1), p2.reshape(-1),
        idx.reshape(-1),
        keys.reshape(-1),
    )
    loss = pl.pallas_call(
        _final_kernel,
        in_specs=[
            pl.BlockSpec((3, _CPAD, _NCAND), lambda: (0, 0, 0)),
            pl.BlockSpec((3, 8, _NCAND), lambda: (0, 0, 0)),
            pl.BlockSpec((3, _NCAND), lambda: (0, 0)),
            pl.BlockSpec(memory_space=pltpu.SMEM),
        ],
        out_specs=pl.BlockSpec(memory_space=pltpu.SMEM),
        out_shape=jax.ShapeDtypeStruct((1,), jnp.float32),
    )(gath.reshape(3, _CPAD, _NCAND), meta.reshape(3, 8, _NCAND),
      win.reshape(3, _NCAND), dense)
    return loss[0]


# speedup vs baseline: 3.0116x; 1.8545x over previous
import functools

import jax
import jax.numpy as jnp
import numpy as np
from jax import lax
from jax.experimental import pallas as pl
from jax.experimental.pallas import tpu as pltpu
from jax.experimental.pallas import tpu_sc as plsc

_NC = 80
_NA = 3
_B = 16
_NT = 320
_NCAND = 5120
_CPAD = 88
_HWS = ((80, 80), (40, 40), (20, 20))
_STRIDES = np.array([8.0, 16.0, 32.0], dtype=np.float32)
_ANCH = (np.array(
    [[10, 13, 16, 30, 33, 23],
     [30, 61, 62, 45, 59, 119],
     [116, 90, 156, 198, 373, 326]], dtype=np.float32)
    .reshape(3, 3, 2) / _STRIDES.reshape(3, 1, 1))
_BAL = (4.0, 1.0, 0.4)
_KEYBASE = (0, 0, 81920)
_DUMPBASE = (307200, 76800, 101120)
_SBUF = 312320
_EPL = _CPAD * _NCAND
_EPT = _EPL // 32
_ROWS = _EPT // 128

_sc_info = None


def _prep_kernel(tT_ref, meta_ref, idx_ref, keys_ref):
    r = lax.broadcasted_iota(jnp.int32, (16, _NT), 0)
    o = r // 3
    a = r % 3
    valid = jnp.where(r < 15, 1.0, 0.0).astype(jnp.float32)
    bf = tT_ref[0:1, :]
    clsf = tT_ref[1:2, :]
    for l in range(3):
        H, W = _HWS[l]
        hw = H * W
        xx = tT_ref[2:3, :] * float(W) + jnp.zeros((16, _NT), jnp.float32)
        yy = tT_ref[3:4, :] * float(H) + jnp.zeros((16, _NT), jnp.float32)
        ww = tT_ref[4:5, :] * float(W) + jnp.zeros((16, _NT), jnp.float32)
        hh = tT_ref[5:6, :] * float(H) + jnp.zeros((16, _NT), jnp.float32)
        aw = jnp.where(a == 0, float(_ANCH[l, 0, 0]),
             jnp.where(a == 1, float(_ANCH[l, 1, 0]), float(_ANCH[l, 2, 0])))
        ah = jnp.where(a == 0, float(_ANCH[l, 0, 1]),
             jnp.where(a == 1, float(_ANCH[l, 1, 1]), float(_ANCH[l, 2, 1])))
        rw = ww / aw
        rh = hh / ah
        mr = jnp.maximum(jnp.maximum(rw, 1.0 / rw), jnp.maximum(rh, 1.0 / rh))
        mself = jnp.where(mr < 4.0, 1.0, 0.0)
        remx = xx - jnp.floor(xx)
        remy = yy - jnp.floor(yy)
        one = jnp.ones((16, _NT), jnp.float32)
        zero = jnp.zeros((16, _NT), jnp.float32)
        lxc = jnp.where(remx < 0.5, one, zero) * jnp.where(xx > 1.0, one, zero)
        lyc = jnp.where(remy < 0.5, one, zero) * jnp.where(yy > 1.0, one, zero)
        gxc = (jnp.where(remx > 0.5, one, zero)
               * jnp.where(xx < float(W) - 1.0, one, zero))
        gyc = (jnp.where(remy > 0.5, one, zero)
               * jnp.where(yy < float(H) - 1.0, one, zero))
        cond = jnp.where(o == 0, one,
               jnp.where(o == 1, lxc,
               jnp.where(o == 2, lyc,
               jnp.where(o == 3, gxc, gyc))))
        maskf = mself * cond * valid
        offx = jnp.where(o == 1, 0.5, jnp.where(o == 3, -0.5, 0.0))
        offy = jnp.where(o == 2, 0.5, jnp.where(o == 4, -0.5, 0.0))
        gridx = jnp.clip((xx - offx).astype(jnp.int32), 0, W - 1)
        gridy = jnp.clip((yy - offy).astype(jnp.int32), 0, H - 1)
        bi = bf.astype(jnp.int32) + jnp.zeros((16, _NT), jnp.int32)
        meta_ref[l, 0] = maskf
        meta_ref[l, 1] = xx - gridx.astype(jnp.float32)
        meta_ref[l, 2] = yy - gridy.astype(jnp.float32)
        meta_ref[l, 3] = ww
        meta_ref[l, 4] = hh
        meta_ref[l, 5] = aw + jnp.zeros((16, _NT), jnp.float32)
        meta_ref[l, 6] = ah + jnp.zeros((16, _NT), jnp.float32)
        meta_ref[l, 7] = clsf + jnp.zeros((16, _NT), jnp.float32)
        del hw
        base = ((bi * H + gridy) * W + gridx) * 255 + a * 85
        cix = lax.broadcasted_iota(jnp.int32, (_CPAD, 16, _NT), 0)
        cm = jnp.where(cix < 85, cix, cix - 85)
        idx_ref[l] = base[None, :, :] + cm
        flat = r * _NT + lax.broadcasted_iota(jnp.int32, (16, _NT), 1)
        pos = ((bi * 3 + a) * H + gridy) * W + gridx + _KEYBASE[l]
        keys_ref[l] = jnp.where(maskf > 0.0, pos, _DUMPBASE[l] + flat)


def _dense_kernel(p0_ref, p1_ref, p2_ref, out_ref):
    i = pl.program_id(0)

    @pl.when(i == 0)
    def _():
        for l in range(3):
            out_ref[l] = 0.0

    for l, ref in enumerate((p0_ref, p1_ref, p2_ref)):
        s = 0.0
        for ch in (4, 89, 174):
            x = ref[:, ch:ch + 1]
            s += jnp.sum(jnp.maximum(x, 0.0)
                         + jnp.log1p(jnp.exp(-jnp.abs(x))))
        out_ref[l] += s


def _sc_body(pf0, pf1, pf2, idx_hbm, keys_hbm, gath_hbm, win_hbm,
             idxv, gbuf, kv, idv, wv, sbuf, sem):
    c = lax.axis_index("c")
    s = lax.axis_index("s")
    wid = s * 2 + c
    for l, pf in enumerate((pf0, pf1, pf2)):
        pltpu.sync_copy(
            idx_hbm.at[pl.ds(l * _EPL + wid * _EPT, _EPT)], idxv)
        pltpu.async_copy(pf.at[idxv], gbuf, sem).wait()
        pltpu.sync_copy(
            gbuf, gath_hbm.at[pl.ds(l * _EPL + wid * _EPT, _EPT)])
    for q in range(4):
        for k in range(5):
            idv[q, pl.ds(k * 16, 16)] = (
                s * 320 + q * 80 + k * 16 + lax.iota(jnp.int32, 16))
    my_layers = ((0,), (1, 2))
    for ci in range(2):
        @pl.when(c == ci)
        def _():
            for l in my_layers[ci]:
                for j in range(4):
                    pltpu.sync_copy(
                        keys_hbm.at[pl.ds(l * _NCAND + s * 320 + j * 80, 80)],
                        kv.at[j])
                for j in range(4):
                    pltpu.sync_copy(idv.at[j], sbuf.at[kv.at[j]])
    plsc.subcore_barrier()
    for ci in range(2):
        @pl.when(c == ci)
        def _():
            for l in my_layers[ci]:
                for j in range(4):
                    pltpu.sync_copy(
                        keys_hbm.at[pl.ds(l * _NCAND + s * 320 + j * 80, 80)],
                        kv.at[j])
                for j in range(4):
                    pltpu.sync_copy(sbuf.at[kv.at[j]], wv.at[j])
                for j in range(4):
                    pltpu.sync_copy(
                        wv.at[j],
                        win_hbm.at[pl.ds(l * _NCAND + s * 320 + j * 80, 80)])


_sc_gather = functools.partial(
    pl.kernel,
    out_type=(jax.ShapeDtypeStruct((3 * _EPL,), jnp.float32),
              jax.ShapeDtypeStruct((3 * _NCAND,), jnp.int32)),
    mesh=plsc.VectorSubcoreMesh(core_axis_name="c", subcore_axis_name="s"),
    scratch_types=(
        pltpu.VMEM((_EPT,), jnp.int32),
        pltpu.VMEM((_EPT,), jnp.float32),
        pltpu.VMEM((4, 80), jnp.int32),
        pltpu.VMEM((4, 80), jnp.int32),
        pltpu.VMEM((4, 80), jnp.int32),
        pltpu.VMEM_SHARED((_SBUF,), jnp.int32),
        pltpu.SemaphoreType.DMA,
    ),
)(_sc_body)


def _final_kernel(gath_ref, meta_ref, win_ref, dense_ref, out_ref):
    lbox = 0.0
    lobj = 0.0
    lcls = 0.0
    cid = lax.broadcasted_iota(jnp.int32, (1, _NCAND), 1)
    crow = lax.broadcasted_iota(jnp.int32, (_NC, _NCAND), 0)
    for l in range(3):
        H, W = _HWS[l]
        mask = meta_ref[l, 0:1, :]
        gxr = meta_ref[l, 1:2, :]
        gyr = meta_ref[l, 2:3, :]
        gw = meta_ref[l, 3:4, :]
        gh = meta_ref[l, 4:5, :]
        aw = meta_ref[l, 5:6, :]
        ah = meta_ref[l, 6:7, :]
        clsf = meta_ref[l, 7:8, :]
        px = jax.nn.sigmoid(gath_ref[l, 0:1, :]) * 2.0 - 0.5
        py = jax.nn.sigmoid(gath_ref[l, 1:2, :]) * 2.0 - 0.5
        pw = jnp.square(jax.nn.sigmoid(gath_ref[l, 2:3, :]) * 2.0) * aw
        ph = jnp.square(jax.nn.sigmoid(gath_ref[l, 3:4, :]) * 2.0) * ah
        a_xmin = px - (pw - 1.0) / 2.0
        a_xmax = px + (pw - 1.0) / 2.0
        a_ymin = py - (ph - 1.0) / 2.0
        a_ymax = py + (ph - 1.0) / 2.0
        b_xmin = gxr - (gw - 1.0) / 2.0
        b_xmax = gxr + (gw - 1.0) / 2.0
        b_ymin = gyr - (gh - 1.0) / 2.0
        b_ymax = gyr + (gh - 1.0) / 2.0
        iw = jnp.clip(jnp.minimum(a_xmax, b_xmax)
                      - jnp.maximum(a_xmin, b_xmin) + 1.0, 0.0)
        ih = jnp.clip(jnp.minimum(a_ymax, b_ymax)
                      - jnp.maximum(a_ymin, b_ymin) + 1.0, 0.0)
        inter = iw * ih
        union = pw * ph + gw * gh - inter
        iou = inter / union
        cw = jnp.maximum(a_xmax, b_xmax) - jnp.minimum(a_xmin, b_xmin) + 1.0
        ch = jnp.maximum(a_ymax, b_ymax) - jnp.minimum(a_ymin, b_ymin) + 1.0
        ca = cw * ch
        giou = iou - (ca - union) / ca
        cnt = jnp.sum(mask)
        box_sum = jnp.sum(mask * (1.0 - giou))
        lbox += jnp.where(cnt > 0.0, box_sum / jnp.maximum(cnt, 1.0), 0.0)
        lobj_logit = gath_ref[l, 4:5, :]
        winm = ((win_ref[l:l + 1, :] == cid) & (mask > 0.0)).astype(jnp.float32)
        corr = jnp.sum(winm * lobj_logit * jnp.clip(giou, 0.0))
        lobj += (dense_ref[l] - corr) / float(_B * _NA * H * W) * _BAL[l]
        ocls = gath_ref[l, 5:85, :]
        ct = (crow == clsf.astype(jnp.int32)).astype(jnp.float32)
        bce = (jnp.maximum(ocls, 0.0) - ocls * ct
               + jnp.log1p(jnp.exp(-jnp.abs(ocls))))
        cls_sum = jnp.sum(bce * mask)
        lcls += jnp.where(cnt > 0.0,
                          cls_sum / (jnp.maximum(cnt, 1.0) * float(_NC)), 0.0)
    out_ref[0] = (lbox * 0.05 + lobj * 1.0 + lcls * 0.5) * float(_B)


def kernel(p0, p1, p2, targets):
    tT = targets.T
    pt0 = p0.transpose(0, 2, 3, 1).reshape(16 * 80 * 80, 255)
    pt1 = p1.transpose(0, 2, 3, 1).reshape(16 * 40 * 40, 255)
    pt2 = p2.transpose(0, 2, 3, 1).reshape(16 * 20 * 20, 255)
    meta, idx, keys = pl.pallas_call(
        _prep_kernel,
        out_shape=(
            jax.ShapeDtypeStruct((3, 8, 16, _NT), jnp.float32),
            jax.ShapeDtypeStruct((3, _CPAD, 16, _NT), jnp.int32),
            jax.ShapeDtypeStruct((3, 16, _NT), jnp.int32),
        ),
    )(tT)
    dense = pl.pallas_call(
        _dense_kernel,
        grid=(100,),
        in_specs=[
            pl.BlockSpec((1024, 255), lambda i: (i, 0)),
            pl.BlockSpec((256, 255), lambda i: (i, 0)),
            pl.BlockSpec((64, 255), lambda i: (i, 0)),
        ],
        out_specs=pl.BlockSpec(memory_space=pltpu.SMEM),
        out_shape=jax.ShapeDtypeStruct((3,), jnp.float32),
    )(pt0, pt1, pt2)
    gath, win = _sc_gather(
        pt0.reshape(-1), pt1.reshape(-1), pt2.reshape(-1),
        idx.reshape(-1),
        keys.reshape(-1),
    )
    loss = pl.pallas_call(
        _final_kernel,
        in_specs=[
            pl.BlockSpec((3, _CPAD, _NCAND), lambda: (0, 0, 0)),
            pl.BlockSpec((3, 8, _NCAND), lambda: (0, 0, 0)),
            pl.BlockSpec((3, _NCAND), lambda: (0, 0)),
            pl.BlockSpec(memory_space=pltpu.SMEM),
        ],
        out_specs=pl.BlockSpec(memory_space=pltpu.SMEM),
        out_shape=jax.ShapeDtypeStruct((1,), jnp.float32),
    )(gath.reshape(3, _CPAD, _NCAND), meta.reshape(3, 8, _NCAND),
      win.reshape(3, _NCAND), dense)
    return loss[0]
